# Initial kernel scaffold; baseline (speedup 1.0000x reference)
import jax
import jax.numpy as jnp
from jax.experimental import pallas as pl


def kernel(input_ids, attention_mask, token_type_ids, pixel_values, pixel_mask, word_emb, text_pos_emb, text_type_emb, ln_w, ln_b, mod_type_emb, patch_w, patch_b, cls_token, vis_pos_emb, img_pos_emb):
    raise NotImplementedError("write your pallas kernel here")


# speedup vs baseline: 1.5374x; 1.5374x over previous
import functools

import jax
import jax.numpy as jnp
from jax import lax
from jax.experimental import pallas as pl
from jax.experimental.pallas import tpu as pltpu
from jax.experimental.pallas import tpu_sc as plsc

D = 768
P = 32
GRID = 12
NPATCH = GRID * GRID
VSEQ = NPATCH + 1
EPS = 1e-12


def _sc_gather(table, idx):
    info = plsc.get_sparse_core_info()
    nw = info.num_cores * info.num_subcores
    ntok = idx.shape[0]
    npad = ((ntok + 8 * nw - 1) // (8 * nw)) * (8 * nw)
    idx_pad = jnp.concatenate(
        [idx, jnp.zeros((npad - ntok,), jnp.int32)])
    b_per_w = npad // nw
    mesh = plsc.VectorSubcoreMesh(core_axis_name="c", subcore_axis_name="s")

    @functools.partial(
        pl.kernel,
        mesh=mesh,
        out_type=jax.ShapeDtypeStruct((npad, D), jnp.float32),
        scratch_types=[
            pltpu.VMEM((b_per_w,), jnp.int32),
            pltpu.VMEM((b_per_w, D), jnp.float32),
            pltpu.SemaphoreType.DMA,
        ],
    )
    def k(table_hbm, idx_hbm, out_hbm, idx_v, rows_v, sem):
        wid = lax.axis_index("s") * info.num_cores + lax.axis_index("c")
        base = wid * b_per_w
        pltpu.sync_copy(idx_hbm.at[pl.ds(base, b_per_w)], idx_v)
        pltpu.async_copy(table_hbm.at[idx_v], rows_v, sem).wait()
        pltpu.sync_copy(rows_v, out_hbm.at[pl.ds(base, b_per_w)])

    return k(table, idx_pad)[:ntok]


def _text_body(rows_ref, tti_ref, pos_ref, type_ref, lnw_ref, lnb_ref,
               mod_ref, out_ref):
    x = rows_ref[0]
    t = tti_ref[0]
    t_col = jnp.transpose(t)
    type0 = type_ref[0:1, :]
    type1 = type_ref[1:2, :]
    tte = type0 + t_col * (type1 - type0)
    x = x + pos_ref[...] + tte
    mu = jnp.mean(x, axis=-1, keepdims=True)
    var = jnp.mean((x - mu) ** 2, axis=-1, keepdims=True)
    y = (x - mu) * lax.rsqrt(var + EPS) * lnw_ref[...] + lnb_ref[...]
    out_ref[0] = y + mod_ref[0:1, :]


def _text_call(rows, tti_f, text_pos_emb, text_type_emb, ln_w, ln_b,
               mod_type_emb):
    b, l, _ = rows.shape
    return pl.pallas_call(
        _text_body,
        grid=(b,),
        in_specs=[
            pl.BlockSpec((1, l, D), lambda i: (i, 0, 0)),
            pl.BlockSpec((1, 1, l), lambda i: (i, 0, 0)),
            pl.BlockSpec((l, D), lambda i: (0, 0)),
            pl.BlockSpec((2, D), lambda i: (0, 0)),
            pl.BlockSpec((1, D), lambda i: (0, 0)),
            pl.BlockSpec((1, D), lambda i: (0, 0)),
            pl.BlockSpec((2, D), lambda i: (0, 0)),
        ],
        out_specs=pl.BlockSpec((1, l, D), lambda i: (i, 0, 0)),
        out_shape=jax.ShapeDtypeStruct((b, l, D), jnp.float32),
    )(rows, tti_f, text_pos_emb, text_type_emb, ln_w.reshape(1, D),
      ln_b.reshape(1, D), mod_type_emb)


def _vis_body(pv_ref, wt_ref, bias_ref, cls_ref, vis_pos_ref, img_pos_ref,
              mod_ref, out_ref):
    x = pv_ref[0]
    x = x.reshape(3, GRID, P, GRID, P)
    x = jnp.transpose(x, (1, 3, 0, 2, 4))
    x = x.reshape(NPATCH, 3 * P * P)
    y = jnp.dot(x, wt_ref[...], preferred_element_type=jnp.float32)
    mod1 = mod_ref[1:2, :]
    ip = img_pos_ref[0]
    y = y + bias_ref[...] + vis_pos_ref[1:VSEQ, :] + mod1 + ip[1:VSEQ, :]
    row0 = cls_ref[...] + vis_pos_ref[0:1, :] + mod1 + ip[0:1, :]
    out_ref[0] = jnp.concatenate([row0, y], axis=0)


def _vis_call(pv, wt, patch_b, cls, vis_pos, img_pos, mod_type_emb):
    n = pv.shape[0]
    s = img_pos.shape[0]
    return pl.pallas_call(
        _vis_body,
        grid=(n,),
        in_specs=[
            pl.BlockSpec((1, 3, P * GRID, P * GRID), lambda i: (i, 0, 0, 0)),
            pl.BlockSpec((3 * P * P, D), lambda i: (0, 0)),
            pl.BlockSpec((1, D), lambda i: (0, 0)),
            pl.BlockSpec((1, D), lambda i: (0, 0)),
            pl.BlockSpec((VSEQ, D), lambda i: (0, 0)),
            pl.BlockSpec((1, VSEQ, D), lambda i: (i % s, 0, 0)),
            pl.BlockSpec((2, D), lambda i: (0, 0)),
        ],
        out_specs=pl.BlockSpec((1, VSEQ, D), lambda i: (i, 0, 0)),
        out_shape=jax.ShapeDtypeStruct((n, VSEQ, D), jnp.float32),
    )(pv, wt, patch_b.reshape(1, D), cls.reshape(1, D),
      vis_pos.reshape(VSEQ, D), img_pos, mod_type_emb)


def kernel(input_ids, attention_mask, token_type_ids, pixel_values,
           pixel_mask, word_emb, text_pos_emb, text_type_emb, ln_w, ln_b,
           mod_type_emb, patch_w, patch_b, cls_token, vis_pos_emb,
           img_pos_emb):
    b, l = input_ids.shape
    s = pixel_values.shape[1]
    n = b * s

    rows = _sc_gather(word_emb, input_ids.reshape(-1)).reshape(b, l, D)

    tti_f = token_type_ids.astype(jnp.float32).reshape(b, 1, l)
    te = _text_call(rows, tti_f, text_pos_emb, text_type_emb, ln_w, ln_b,
                    mod_type_emb)

    wt = patch_w.reshape(D, 3 * P * P).T
    pv = pixel_values.reshape(n, 3, P * GRID, P * GRID)
    vis = _vis_call(pv, wt, patch_b, cls_token, vis_pos_emb, img_pos_emb,
                    mod_type_emb)
    vis = vis.reshape(b, s * VSEQ, D)

    embeddings = jnp.concatenate([te, vis], axis=1)

    m = pixel_mask[:, :, ::P, ::P].reshape(b, s, NPATCH)
    vm = jnp.concatenate([jnp.ones((b, s, 1), m.dtype), m],
                         axis=2).reshape(b, s * VSEQ)
    masks = jnp.concatenate([attention_mask, vm], axis=1)
    return embeddings, masks


# speedup vs baseline: 1.7254x; 1.1223x over previous
import functools

import jax
import jax.numpy as jnp
from jax import lax
from jax.experimental import pallas as pl
from jax.experimental.pallas import tpu as pltpu
from jax.experimental.pallas import tpu_sc as plsc

D = 768
P = 32
GRID = 12
NPATCH = GRID * GRID
VSEQ = NPATCH + 1
EPS = 1e-12


def _sc_gather(table, idx):
    info = plsc.get_sparse_core_info()
    nw = info.num_cores * info.num_subcores
    ntok = idx.shape[0]
    npad = ((ntok + 8 * nw - 1) // (8 * nw)) * (8 * nw)
    idx_pad = jnp.concatenate(
        [idx, jnp.zeros((npad - ntok,), jnp.int32)])
    b_per_w = npad // nw
    mesh = plsc.VectorSubcoreMesh(core_axis_name="c", subcore_axis_name="s")

    @functools.partial(
        pl.kernel,
        mesh=mesh,
        out_type=jax.ShapeDtypeStruct((npad, D), jnp.float32),
        scratch_types=[
            pltpu.VMEM((b_per_w,), jnp.int32),
            pltpu.VMEM((b_per_w, D), jnp.float32),
            pltpu.SemaphoreType.DMA,
        ],
    )
    def k(table_hbm, idx_hbm, out_hbm, idx_v, rows_v, sem):
        wid = lax.axis_index("s") * info.num_cores + lax.axis_index("c")
        base = wid * b_per_w
        pltpu.sync_copy(idx_hbm.at[pl.ds(base, b_per_w)], idx_v)
        pltpu.async_copy(table_hbm.at[idx_v], rows_v, sem).wait()
        pltpu.sync_copy(rows_v, out_hbm.at[pl.ds(base, b_per_w)])

    return k(table, idx_pad)[:ntok]


def _text_body(rows_ref, tti_ref, pos_ref, type_ref, lnw_ref, lnb_ref,
               mod_ref, out_ref):
    x = rows_ref[0]
    t = tti_ref[0]
    t_col = jnp.transpose(t)
    type0 = type_ref[0:1, :]
    type1 = type_ref[1:2, :]
    tte = type0 + t_col * (type1 - type0)
    x = x + pos_ref[...] + tte
    mu = jnp.mean(x, axis=-1, keepdims=True)
    var = jnp.mean((x - mu) ** 2, axis=-1, keepdims=True)
    y = (x - mu) * lax.rsqrt(var + EPS) * lnw_ref[...] + lnb_ref[...]
    out_ref[0] = y + mod_ref[0:1, :]


def _text_call(rows, tti_f, text_pos_emb, text_type_emb, ln_w, ln_b,
               mod_type_emb):
    b, l, _ = rows.shape
    return pl.pallas_call(
        _text_body,
        grid=(b,),
        in_specs=[
            pl.BlockSpec((1, l, D), lambda i: (i, 0, 0)),
            pl.BlockSpec((1, 1, l), lambda i: (i, 0, 0)),
            pl.BlockSpec((l, D), lambda i: (0, 0)),
            pl.BlockSpec((2, D), lambda i: (0, 0)),
            pl.BlockSpec((1, D), lambda i: (0, 0)),
            pl.BlockSpec((1, D), lambda i: (0, 0)),
            pl.BlockSpec((2, D), lambda i: (0, 0)),
        ],
        out_specs=pl.BlockSpec((1, l, D), lambda i: (i, 0, 0)),
        out_shape=jax.ShapeDtypeStruct((b, l, D), jnp.float32),
    )(rows, tti_f, text_pos_emb, text_type_emb, ln_w.reshape(1, D),
      ln_b.reshape(1, D), mod_type_emb)


def _vis_body(pv_ref, wt_ref, bias_ref, cls_ref, vis_pos_ref, img_pos_ref,
              mod_ref, out_ref):
    x = pv_ref[0].astype(jnp.bfloat16)
    m = x.reshape(3 * GRID * P, GRID * P)
    mt = jnp.transpose(m)
    mt = mt.reshape(GRID, P, 3 * GRID * P)
    mt = jnp.transpose(mt, (0, 2, 1))
    mt = mt.reshape(GRID, 3, GRID, P, P)
    mt = jnp.transpose(mt, (0, 2, 1, 3, 4))
    xall = mt.reshape(NPATCH, 3 * P * P)
    y = lax.dot_general(xall, wt_ref[...],
                        (((1,), (1,)), ((), ())),
                        preferred_element_type=jnp.float32)
    y = y.reshape(GRID, GRID, D)
    y = jnp.transpose(y, (1, 0, 2)).reshape(NPATCH, D)
    mod1 = mod_ref[1:2, :]
    ip = img_pos_ref[0]
    y = y + bias_ref[...] + vis_pos_ref[1:VSEQ, :] + mod1 + ip[1:VSEQ, :]
    row0 = cls_ref[...] + vis_pos_ref[0:1, :] + mod1 + ip[0:1, :]
    out_ref[0] = jnp.concatenate([row0, y], axis=0)


def _vis_call(pv, wt, patch_b, cls, vis_pos, img_pos, mod_type_emb):
    n = pv.shape[0]
    s = img_pos.shape[0]
    return pl.pallas_call(
        _vis_body,
        grid=(n,),
        in_specs=[
            pl.BlockSpec((1, 3, P * GRID, P * GRID), lambda i: (i, 0, 0, 0)),
            pl.BlockSpec((D, 3 * P * P), lambda i: (0, 0)),
            pl.BlockSpec((1, D), lambda i: (0, 0)),
            pl.BlockSpec((1, D), lambda i: (0, 0)),
            pl.BlockSpec((VSEQ, D), lambda i: (0, 0)),
            pl.BlockSpec((1, VSEQ, D), lambda i: (i % s, 0, 0)),
            pl.BlockSpec((2, D), lambda i: (0, 0)),
        ],
        out_specs=pl.BlockSpec((1, VSEQ, D), lambda i: (i, 0, 0)),
        out_shape=jax.ShapeDtypeStruct((n, VSEQ, D), jnp.float32),
    )(pv, wt, patch_b.reshape(1, D), cls.reshape(1, D),
      vis_pos.reshape(VSEQ, D), img_pos, mod_type_emb)


def kernel(input_ids, attention_mask, token_type_ids, pixel_values,
           pixel_mask, word_emb, text_pos_emb, text_type_emb, ln_w, ln_b,
           mod_type_emb, patch_w, patch_b, cls_token, vis_pos_emb,
           img_pos_emb):
    b, l = input_ids.shape
    s = pixel_values.shape[1]
    n = b * s

    rows = _sc_gather(word_emb, input_ids.reshape(-1)).reshape(b, l, D)

    tti_f = token_type_ids.astype(jnp.float32).reshape(b, 1, l)
    te = _text_call(rows, tti_f, text_pos_emb, text_type_emb, ln_w, ln_b,
                    mod_type_emb)

    wt = patch_w.reshape(D, 3 * P * P).astype(jnp.bfloat16)
    pv = pixel_values.reshape(n, 3, P * GRID, P * GRID)
    vis = _vis_call(pv, wt, patch_b, cls_token, vis_pos_emb, img_pos_emb,
                    mod_type_emb)
    vis = vis.reshape(b, s * VSEQ, D)

    embeddings = jnp.concatenate([te, vis], axis=1)

    m = pixel_mask[:, :, ::P, ::P].reshape(b, s, NPATCH)
    vm = jnp.concatenate([jnp.ones((b, s, 1), m.dtype), m],
                         axis=2).reshape(b, s * VSEQ)
    masks = jnp.concatenate([attention_mask, vm], axis=1)
    return embeddings, masks


# speedup vs baseline: 2.6154x; 1.5159x over previous
import functools

import jax
import jax.numpy as jnp
from jax import lax
from jax.experimental import pallas as pl
from jax.experimental.pallas import tpu as pltpu
from jax.experimental.pallas import tpu_sc as plsc

D = 768
P = 32
GRID = 12
NPATCH = GRID * GRID
VSEQ = NPATCH + 1
EPS = 1e-12


def _sc_gather(table, idx):
    info = plsc.get_sparse_core_info()
    nw = info.num_cores * info.num_subcores
    ntok = idx.shape[0]
    npad = ((ntok + 8 * nw - 1) // (8 * nw)) * (8 * nw)
    idx_pad = jnp.concatenate(
        [idx, jnp.zeros((npad - ntok,), jnp.int32)])
    b_per_w = npad // nw
    mesh = plsc.VectorSubcoreMesh(core_axis_name="c", subcore_axis_name="s")

    @functools.partial(
        pl.kernel,
        mesh=mesh,
        out_type=jax.ShapeDtypeStruct((npad, D), jnp.float32),
        scratch_types=[
            pltpu.VMEM((b_per_w,), jnp.int32),
            pltpu.VMEM((b_per_w, D), jnp.float32),
            pltpu.SemaphoreType.DMA,
        ],
    )
    def k(table_hbm, idx_hbm, out_hbm, idx_v, rows_v, sem):
        wid = lax.axis_index("s") * info.num_cores + lax.axis_index("c")
        base = wid * b_per_w
        pltpu.sync_copy(idx_hbm.at[pl.ds(base, b_per_w)], idx_v)
        pltpu.async_copy(table_hbm.at[idx_v], rows_v, sem).wait()
        pltpu.sync_copy(rows_v, out_hbm.at[pl.ds(base, b_per_w)])

    return k(table, idx_pad)[:ntok]


def _im2col(img):
    m = img.reshape(3 * GRID * P, GRID * P)
    mt = jnp.transpose(m)
    mt = mt.reshape(GRID, P, 3 * GRID * P)
    mt = jnp.transpose(mt, (0, 2, 1))
    mt = mt.reshape(GRID, 3, GRID, P, P)
    mt = jnp.transpose(mt, (0, 2, 1, 3, 4))
    return mt.reshape(NPATCH, 3 * P * P)


def _main_body(pv_ref, rows_ref, tti_ref, tpos_ref, ttype_ref, lnw_ref,
               lnb_ref, mod_ref, wt_ref, bias_ref, cls_ref, vis_pos_ref,
               img_pos_ref, out_ref):
    s_count = pv_ref.shape[1]
    x = rows_ref[0]
    t = tti_ref[0]
    t_col = jnp.transpose(t)
    type0 = ttype_ref[0:1, :]
    type1 = ttype_ref[1:2, :]
    x = x + tpos_ref[...] + type0 + t_col * (type1 - type0)
    mu = jnp.mean(x, axis=-1, keepdims=True)
    var = jnp.mean((x - mu) ** 2, axis=-1, keepdims=True)
    te = (x - mu) * lax.rsqrt(var + EPS) * lnw_ref[...] + lnb_ref[...]
    te = te + mod_ref[0:1, :]
    l = rows_ref.shape[1]
    out_ref[0, 0:l, :] = te

    mod1 = mod_ref[1:2, :]
    for s in range(s_count):
        xs = _im2col(pv_ref[0, s].astype(jnp.bfloat16))
        ys = lax.dot_general(xs, wt_ref[...],
                             (((1,), (1,)), ((), ())),
                             preferred_element_type=jnp.float32)
        ys = ys.reshape(GRID, GRID, D)
        ys = jnp.transpose(ys, (1, 0, 2)).reshape(NPATCH, D)
        ip = img_pos_ref[s]
        ys = ys + bias_ref[...] + vis_pos_ref[1:VSEQ, :] + mod1 + ip[1:VSEQ, :]
        row0 = cls_ref[...] + vis_pos_ref[0:1, :] + mod1 + ip[0:1, :]
        rows_s = jnp.concatenate([row0, ys], axis=0)
        base = l + s * VSEQ
        out_ref[0, base:base + VSEQ, :] = rows_s


def _main_call(pv, rows, tti_f, text_pos_emb, text_type_emb, ln_w, ln_b,
               mod_type_emb, wt, patch_b, cls, vis_pos, img_pos):
    b, s = pv.shape[0], pv.shape[1]
    l = rows.shape[1]
    seq = l + s * VSEQ
    return pl.pallas_call(
        _main_body,
        grid=(b,),
        in_specs=[
            pl.BlockSpec((1, s, 3, P * GRID, P * GRID),
                         lambda i: (i, 0, 0, 0, 0)),
            pl.BlockSpec((1, l, D), lambda i: (i, 0, 0)),
            pl.BlockSpec((1, 1, l), lambda i: (i, 0, 0)),
            pl.BlockSpec((l, D), lambda i: (0, 0)),
            pl.BlockSpec((2, D), lambda i: (0, 0)),
            pl.BlockSpec((1, D), lambda i: (0, 0)),
            pl.BlockSpec((1, D), lambda i: (0, 0)),
            pl.BlockSpec((2, D), lambda i: (0, 0)),
            pl.BlockSpec((D, 3 * P * P), lambda i: (0, 0)),
            pl.BlockSpec((1, D), lambda i: (0, 0)),
            pl.BlockSpec((1, D), lambda i: (0, 0)),
            pl.BlockSpec((VSEQ, D), lambda i: (0, 0)),
            pl.BlockSpec((s, VSEQ, D), lambda i: (0, 0, 0)),
        ],
        out_specs=pl.BlockSpec((1, seq, D), lambda i: (i, 0, 0)),
        out_shape=jax.ShapeDtypeStruct((b, seq, D), jnp.float32),
    )(pv, rows, tti_f, text_pos_emb, text_type_emb, ln_w.reshape(1, D),
      ln_b.reshape(1, D), mod_type_emb, wt, patch_b.reshape(1, D),
      cls.reshape(1, D), vis_pos.reshape(VSEQ, D), img_pos)


def kernel(input_ids, attention_mask, token_type_ids, pixel_values,
           pixel_mask, word_emb, text_pos_emb, text_type_emb, ln_w, ln_b,
           mod_type_emb, patch_w, patch_b, cls_token, vis_pos_emb,
           img_pos_emb):
    b, l = input_ids.shape
    s = pixel_values.shape[1]

    rows = _sc_gather(word_emb, input_ids.reshape(-1)).reshape(b, l, D)

    tti_f = token_type_ids.astype(jnp.float32).reshape(b, 1, l)
    wt = patch_w.reshape(D, 3 * P * P).astype(jnp.bfloat16)
    embeddings = _main_call(pixel_values, rows, tti_f, text_pos_emb,
                            text_type_emb, ln_w, ln_b, mod_type_emb, wt,
                            patch_b, cls_token, vis_pos_emb, img_pos_emb)

    m = pixel_mask[:, :, ::P, ::P].reshape(b, s, NPATCH)
    vm = jnp.concatenate([jnp.ones((b, s, 1), m.dtype), m],
                         axis=2).reshape(b, s * VSEQ)
    masks = jnp.concatenate([attention_mask, vm], axis=1)
    return embeddings, masks
